# baseline (speedup 1.0000x reference)
import functools

import jax
import jax.numpy as jnp
from jax import lax
from jax.experimental import pallas as pl
from jax.experimental.pallas import tpu as pltpu
from jax.experimental.pallas import tpu_sc as plsc

H = 128
NCELL = 12800
NCLU = 1408
NSTEP = 4
NW = 32
CS = 40
_F32 = jnp.float32

GY = ((12800, 'u', 'c', 0),
      (140800, 'u', 'c', 1),
      (2560, 'c', 'c', 2),
      (2560, 'c', 'c', 3),
      (1280, 'u', 'u', 0),
      (1280, 'u', 'u', 1))
GX = ((230400, 'c', 'c', 0),)


def _bases(spec):
    b, acc = [], 0
    for e in spec:
        b.append(acc)
        acc += e[0]
    return tuple(b), acc


GY_BASE, GY_E = _bases(GY)
GX_BASE, GX_E = _bases(GX)


@functools.cache
def _sc_mesh():
    return plsc.VectorSubcoreMesh(core_axis_name="c", subcore_axis_name="s")


def _mm_body(x_ref, w_ref, o_ref):
    o_ref[...] = jnp.dot(x_ref[...], w_ref[...], preferred_element_type=_F32)


def _mm(x, w, blk):
    m, k = x.shape
    n = w.shape[1]
    return pl.pallas_call(
        _mm_body,
        grid=(m // blk,),
        in_specs=[pl.BlockSpec((blk, k), lambda i: (i, 0)),
                  pl.BlockSpec((k, n), lambda i: (0, 0))],
        out_specs=pl.BlockSpec((blk, n), lambda i: (i, 0)),
        out_shape=jax.ShapeDtypeStruct((m, n), _F32),
    )(x, w)


def _mmb_body(x_ref, w_ref, o_ref):
    o_ref[...] = jnp.dot(x_ref[...], w_ref[...],
                         preferred_element_type=_F32).astype(jnp.bfloat16)


def _mmb(x, w, blk):
    m, k = x.shape
    n = w.shape[1]
    out = pl.pallas_call(
        _mmb_body,
        grid=(m // blk,),
        in_specs=[pl.BlockSpec((blk, k), lambda i: (i, 0)),
                  pl.BlockSpec((k, n), lambda i: (0, 0))],
        out_specs=pl.BlockSpec((blk, n), lambda i: (i, 0)),
        out_shape=jax.ShapeDtypeStruct((m, n), jnp.bfloat16),
    )(x, w)
    return out.reshape(m * (n // H), H)


def _g_body(w4_ref, wm_ref, o_ref):
    o_ref[0] = lax.dot_general(w4_ref[0], wm_ref[0], (((1,), (1,)), ((), ())),
                               preferred_element_type=_F32)


def _make_g(w4s, wm):
    return pl.pallas_call(
        _g_body,
        grid=(5,),
        in_specs=[pl.BlockSpec((1, H, H), lambda i: (i, 0, 0)),
                  pl.BlockSpec((1, 4 * H, H), lambda i: (i, 0, 0))],
        out_specs=pl.BlockSpec((1, H, 4 * H), lambda i: (i, 0, 0)),
        out_shape=jax.ShapeDtypeStruct((5, H, 4 * H), _F32),
    )(w4s, wm)


MLP_BLK = 1280


def _widx_y(i):
    w = ((i >= 10).astype(jnp.int32) + (i >= 120) + (i >= 122))
    w = jnp.where(i >= 125, 3, jnp.where(i >= 124, 2, w))
    return w


def _mlp_sel_body(x_ref, w2_ref, w3_ref, o_ref):
    y = jnp.maximum(jnp.dot(x_ref[...], w2_ref[0], preferred_element_type=_F32), 0.0)
    o_ref[...] = jnp.maximum(jnp.dot(y, w3_ref[0], preferred_element_type=_F32), 0.0)


def _mlp_y(y1, w2s, w3s):
    return pl.pallas_call(
        _mlp_sel_body,
        grid=(GY_E // MLP_BLK,),
        in_specs=[pl.BlockSpec((MLP_BLK, H), lambda i: (i, 0)),
                  pl.BlockSpec((1, H, H), lambda i: (_widx_y(i), 0, 0)),
                  pl.BlockSpec((1, H, H), lambda i: (_widx_y(i), 0, 0))],
        out_specs=pl.BlockSpec((MLP_BLK, H), lambda i: (i, 0)),
        out_shape=jax.ShapeDtypeStruct((GY_E, H), _F32),
    )(y1, w2s, w3s)


def _mlp_fix_body(x_ref, w2_ref, w3_ref, o_ref):
    y = jnp.maximum(jnp.dot(x_ref[...], w2_ref[...], preferred_element_type=_F32), 0.0)
    o_ref[...] = jnp.maximum(jnp.dot(y, w3_ref[...], preferred_element_type=_F32), 0.0)


def _mlp_x(y1, w2, w3):
    return pl.pallas_call(
        _mlp_fix_body,
        grid=(GX_E // MLP_BLK,),
        in_specs=[pl.BlockSpec((MLP_BLK, H), lambda i: (i, 0)),
                  pl.BlockSpec((H, H), lambda i: (0, 0)),
                  pl.BlockSpec((H, H), lambda i: (0, 0))],
        out_specs=pl.BlockSpec((MLP_BLK, H), lambda i: (i, 0)),
        out_shape=jax.ShapeDtypeStruct((GX_E, H), _F32),
    )(y1, w2, w3)


def _lstm_body(cc_ref, py_ref, px_ref, h_ref, c_ref, g_ref, whh_ref, wc_ref,
               ho_ref, co_ref, tc_ref):
    acc = cc_ref[...] + jnp.dot(h_ref[...], whh_ref[...], preferred_element_type=_F32)
    for t in range(4):
        p = py_ref[0, t] + py_ref[1, t]
        acc = acc + jnp.dot(p, g_ref[t], preferred_element_type=_F32)
    p = px_ref[0, 0] + px_ref[1, 0]
    acc = acc + jnp.dot(p, g_ref[4], preferred_element_type=_F32)
    i = jax.nn.sigmoid(acc[:, 0:H])
    f = jax.nn.sigmoid(acc[:, H:2 * H])
    g = jnp.tanh(acc[:, 2 * H:3 * H])
    o = jax.nn.sigmoid(acc[:, 3 * H:4 * H])
    cn = f * c_ref[...] + i * g
    hn = o * jnp.tanh(cn)
    ho_ref[...] = hn
    co_ref[...] = cn
    tc_ref[...] = jnp.dot(hn, wc_ref[...], preferred_element_type=_F32)


def _lstm(cellconst, py, px, h, c, g, whhT, wcat_cell):
    blk = 1280
    return pl.pallas_call(
        _lstm_body,
        grid=(NCELL // blk,),
        in_specs=[pl.BlockSpec((blk, 4 * H), lambda i: (i, 0)),
                  pl.BlockSpec((2, 4, blk, H), lambda i: (0, 0, i, 0)),
                  pl.BlockSpec((2, 1, blk, H), lambda i: (0, 0, i, 0)),
                  pl.BlockSpec((blk, H), lambda i: (i, 0)),
                  pl.BlockSpec((blk, H), lambda i: (i, 0)),
                  pl.BlockSpec((5, H, 4 * H), lambda i: (0, 0, 0)),
                  pl.BlockSpec((H, 4 * H), lambda i: (0, 0)),
                  pl.BlockSpec((H, 8 * H), lambda i: (0, 0))],
        out_specs=[pl.BlockSpec((blk, H), lambda i: (i, 0)),
                   pl.BlockSpec((blk, H), lambda i: (i, 0)),
                   pl.BlockSpec((blk, 8 * H), lambda i: (i, 0))],
        out_shape=[jax.ShapeDtypeStruct((NCELL, H), _F32),
                   jax.ShapeDtypeStruct((NCELL, H), _F32),
                   jax.ShapeDtypeStruct((NCELL, 8 * H), _F32)],
    )(cellconst, py, px, h, c, g, whhT, wcat_cell)


def _clu_body(hc_ref, pu_ref, w4_ref, wu_ref, o_ref, tu_ref):
    cm = jnp.dot(pu_ref[0, 0] + pu_ref[1, 0], w4_ref[2], preferred_element_type=_F32)
    cm = cm + jnp.dot(pu_ref[0, 1] + pu_ref[1, 1], w4_ref[3], preferred_element_type=_F32)
    hn = jnp.tanh(hc_ref[...] + cm)
    o_ref[...] = hn
    tu_ref[...] = jnp.dot(hn, wu_ref[...], preferred_element_type=_F32)


def _clu_update(h_clu, pu, w4s, wcat_clu):
    return pl.pallas_call(
        _clu_body,
        in_specs=[pl.BlockSpec((NCLU, H), lambda: (0, 0)),
                  pl.BlockSpec((2, 2, NCLU, H), lambda: (0, 0, 0, 0)),
                  pl.BlockSpec((5, H, H), lambda: (0, 0, 0)),
                  pl.BlockSpec((H, 6 * H), lambda: (0, 0))],
        out_specs=[pl.BlockSpec((NCLU, H), lambda: (0, 0)),
                   pl.BlockSpec((NCLU, 6 * H), lambda: (0, 0))],
        out_shape=[jax.ShapeDtypeStruct((NCLU, H), _F32),
                   jax.ShapeDtypeStruct((NCLU, 6 * H), _F32)],
    )(h_clu, pu, w4s, wcat_clu)


def _logits_body(h0, h1, h2, h3, oe_ref, o_ref):
    oe = oe_ref[0]
    for t, hr in enumerate((h0, h1, h2, h3)):
        o_ref[t, 0] = lax.dot_general(hr[0], oe, (((1,), (1,)), ((), ())),
                                      preferred_element_type=_F32)


def _logits(hs, oe3):
    nb = 100
    hs3 = [h.reshape(128, nb, H) for h in hs]
    out = pl.pallas_call(
        _logits_body,
        grid=(128,),
        in_specs=[pl.BlockSpec((1, nb, H), lambda i: (i, 0, 0)) for _ in range(4)]
        + [pl.BlockSpec((1, 11, H), lambda i: (i, 0, 0))],
        out_specs=pl.BlockSpec((4, 1, nb, 11), lambda i: (0, i, 0, 0)),
        out_shape=jax.ShapeDtypeStruct((4, 128, nb, 11), _F32),
    )(*hs3, oe3)
    return out.reshape(4, NCELL, 11)


def _sc_gather(spec, bases, etot, tcell, tclu, gidxa, gidxb):
    pws = tuple(e[0] // NW for e in spec)
    pwmax = max(pws)
    GCH = 128

    @functools.partial(
        pl.kernel,
        out_type=jax.ShapeDtypeStruct((etot, H), _F32),
        mesh=_sc_mesh(),
        scratch_types=[
            pltpu.VMEM((pwmax,), jnp.int32),
            pltpu.VMEM((pwmax,), jnp.int32),
            pltpu.VMEM((2, GCH, H), _F32),
            pltpu.VMEM((2, GCH, H), _F32),
            pltpu.SemaphoreType.DMA,
            pltpu.SemaphoreType.DMA,
            pltpu.SemaphoreType.DMA,
        ],
    )
    def k(tc_ref, tu_ref, ia_ref, ib_ref, y_ref, iav, ibv, ra, rb,
          sema, semb, semw):
        w = lax.axis_index("s") * 2 + lax.axis_index("c")
        for s in range(len(spec)):
            pw = pws[s]
            wbase = bases[s] + w * pw
            ta = tu_ref if spec[s][1] == 'u' else tc_ref
            tb = tu_ref if spec[s][2] == 'u' else tc_ref
            pltpu.sync_copy(ia_ref.at[pl.ds(wbase, pw)], iav.at[pl.ds(0, pw)])
            pltpu.sync_copy(ib_ref.at[pl.ds(wbase, pw)], ibv.at[pl.ds(0, pw)])

            def half(off, u, gc, ta=ta, tb=tb):
                cpa = pltpu.async_copy(ta.at[iav.at[pl.ds(off, gc)]],
                                       ra.at[u, pl.ds(0, gc)], sema)
                cpb = pltpu.async_copy(tb.at[ibv.at[pl.ds(off, gc)]],
                                       rb.at[u, pl.ds(0, gc)], semb)
                return cpa, cpb

            def work(off, u, gc, wbase=wbase):
                def row(r, _):
                    for kk in range(8):
                        a = ra[u, r, pl.ds(kk * 16, 16)]
                        b = rb[u, r, pl.ds(kk * 16, 16)]
                        ra[u, r, pl.ds(kk * 16, 16)] = jnp.maximum(a + b, 0.0)
                    return 0

                lax.fori_loop(0, gc, row, 0)
                return pltpu.async_copy(ra.at[u, pl.ds(0, gc)],
                                        y_ref.at[pl.ds(wbase + off, gc)],
                                        semw)

            def single(off, gc):
                ca, cb = half(off, 0, gc)
                ca.wait()
                cb.wait()
                work(off, 0, gc).wait()

            nfull = pw // GCH
            rem = pw % GCH
            if nfull >= 2:
                def pair(jj, _):
                    j = jj * 2
                    ca0, cb0 = half(j * GCH, 0, GCH)
                    ca1, cb1 = half((j + 1) * GCH, 1, GCH)
                    ca0.wait()
                    cb0.wait()
                    w0 = work(j * GCH, 0, GCH)
                    ca1.wait()
                    cb1.wait()
                    w1 = work((j + 1) * GCH, 1, GCH)
                    w0.wait()
                    w1.wait()
                    return 0

                lax.fori_loop(0, nfull // 2, pair, 0)
            if nfull % 2 == 1:
                single((nfull - 1) * GCH, GCH)
            if rem:
                single(nfull * GCH, rem)

    return k(tcell, tclu, gidxa, gidxb)


def _sc_scatter(spec, bases, ncslot, nuslot, y3, gdst, zc):
    pws = tuple(e[0] // NW for e in spec)
    out_type = [jax.ShapeDtypeStruct((2, ncslot, NCELL, H), _F32)]
    if nuslot:
        out_type.append(jax.ShapeDtypeStruct((2, nuslot, NCLU, H), _F32))

    @functools.partial(
        pl.kernel,
        out_type=out_type,
        mesh=_sc_mesh(),
        scratch_types=[
            pltpu.VMEM((1, 2 * CS), jnp.int32),
            pltpu.VMEM((1, CS), jnp.int32),
            pltpu.VMEM((2 * CS, H), _F32),
            pltpu.VMEM_SHARED((NCELL, H), _F32),
            pltpu.SemaphoreType.DMA,
            pltpu.SemaphoreType.DMA,
            pltpu.SemaphoreType.DMA,
        ],
    )
    def k(y_ref, gd_ref, zc_ref, oc_ref, *rest):
        if nuslot:
            ou_ref = rest[0]
            rest = rest[1:]
        idx8, idx4, ybuf, accum, lsem, isem, ssem = rest
        c = lax.axis_index("c")
        s = lax.axis_index("s")

        for seg in range(len(spec)):
            slot_cell = spec[seg][2] == 'c'
            nrows = NCELL if slot_cell else NCLU
            rpt = nrows // 16
            r0 = s * rpt
            pw = pws[seg]
            cs = 2 * CS if pw % (2 * CS) == 0 else CS
            nch = pw // cs
            off0 = bases[seg] + c * (spec[seg][0] // 2) + s * pw
            pltpu.sync_copy(zc_ref.at[pl.ds(0, rpt)], accum.at[pl.ds(r0, rpt)])
            plsc.subcore_barrier()

            def chunk(j, _, off0=off0, cs=cs):
                o = off0 + j * cs
                iv = idx8 if cs == 2 * CS else idx4
                ly = pltpu.async_copy(y_ref.at[pl.ds(o, cs)],
                                      ybuf.at[pl.ds(0, cs)], lsem)
                l0 = pltpu.async_copy(gd_ref.at[pl.ds(o, cs)], iv.at[0], isem)
                ly.wait()
                l0.wait()
                pltpu.sync_copy(ybuf.at[pl.ds(0, cs)], accum.at[iv.at[0]],
                                add=True)
                return 0

            if nch == 1:
                chunk(0, 0)
            else:
                lax.fori_loop(0, nch, chunk, 0)
            plsc.subcore_barrier()
            slot = spec[seg][3]
            if slot_cell:
                pltpu.sync_copy(accum.at[pl.ds(r0, rpt)],
                                oc_ref.at[c, slot, pl.ds(r0, rpt)])
            else:
                pltpu.sync_copy(accum.at[pl.ds(r0, rpt)],
                                ou_ref.at[c, slot, pl.ds(r0, rpt)])
            plsc.subcore_barrier()

    return k(y3, gdst, zc)


def kernel(cell_x, cluster_x, output_embeddings, params, contains_src,
           contains_dst, may_src, may_dst, lt_edges, gt_edges, diff_edges,
           clt_edges, cgt_edges):
    order = ('contains', 'may_contain', 'lt', 'gt', 'diff')
    w1a = {t: params[t]['W1'][:H] for t in order}
    w1b = {t: params[t]['W1'][H:] for t in order}
    wcat_cell = jnp.concatenate(
        [w1b['contains'], w1b['may_contain'], w1a['lt'], w1b['lt'],
         w1a['gt'], w1b['gt'], w1a['diff'], w1b['diff']], axis=1)
    wcat_clu = jnp.concatenate(
        [w1a['contains'], w1a['may_contain'], w1a['lt'], w1b['lt'],
         w1a['gt'], w1b['gt']], axis=1)
    w2s = jnp.stack([params[t]['W2'] for t in order[:4]])
    w3s = jnp.stack([params[t]['W3'] for t in order[:4]])
    w4s = jnp.stack([params[t]['W4'] for t in order])
    wih = params['Wih']
    wihxT = wih[:, :H].T
    wm = wih[:, H:].reshape(4 * H, 5, H).transpose(1, 0, 2)
    whhT = params['Whh'].T

    i32 = jnp.int32
    gay = jnp.concatenate([
        contains_src.astype(i32) * 6, may_src.astype(i32) * 6 + 1,
        lt_edges[0].astype(i32) * 8 + 2, gt_edges[0].astype(i32) * 8 + 4,
        clt_edges[0].astype(i32) * 6 + 2, cgt_edges[0].astype(i32) * 6 + 4])
    gby = jnp.concatenate([
        contains_dst.astype(i32) * 8, may_dst.astype(i32) * 8 + 1,
        lt_edges[1].astype(i32) * 8 + 3, gt_edges[1].astype(i32) * 8 + 5,
        clt_edges[1].astype(i32) * 6 + 3, cgt_edges[1].astype(i32) * 6 + 5])
    gdy = jnp.concatenate([
        contains_dst.astype(i32), may_dst.astype(i32), lt_edges[1].astype(i32),
        gt_edges[1].astype(i32), clt_edges[1].astype(i32),
        cgt_edges[1].astype(i32)])
    gax = diff_edges[0].astype(i32) * 8 + 6
    gbx = diff_edges[1].astype(i32) * 8 + 7
    gdx = diff_edges[1].astype(i32)

    cellconst = _mm(cell_x, wihxT, 1280)
    g = _make_g(w4s, wm)
    zc = jnp.zeros((NCELL // 16, H), _F32)

    h_cell = cell_x
    h_clu = cluster_x
    rnn_h = jnp.zeros((NCELL, H), _F32)
    rnn_c = jnp.zeros((NCELL, H), _F32)
    hs = []
    tcell = _mm(cell_x, wcat_cell, 1280).reshape(NCELL * 8, H)
    tclu = _mm(cluster_x, wcat_clu, NCLU).reshape(NCLU * 6, H)
    for step in range(NSTEP):
        y1y = _sc_gather(GY, GY_BASE, GY_E, tcell, tclu, gay, gby)
        y1x = _sc_gather(GX, GX_BASE, GX_E, tcell, tclu, gax, gbx)
        y3y = _mlp_y(y1y, w2s, w3s)
        y3x = _mlp_x(y1x, params['diff']['W2'], params['diff']['W3'])
        py, pu = _sc_scatter(GY, GY_BASE, 4, 2, y3y, gdy, zc)
        (px,) = _sc_scatter(GX, GX_BASE, 1, 0, y3x, gdx, zc)
        rnn_h, rnn_c, tcell2 = _lstm(cellconst, py, px, rnn_h, rnn_c, g,
                                     whhT, wcat_cell)
        h_cell = rnn_h
        h_clu, tclu2 = _clu_update(h_clu, pu, w4s, wcat_clu)
        tcell = tcell2.reshape(NCELL * 8, H)
        tclu = tclu2.reshape(NCLU * 6, H)
        hs.append(h_cell)

    oe3 = output_embeddings.reshape(128, 11, H)
    return _logits(hs, oe3)


# speedup vs baseline: 1.0039x; 1.0039x over previous
import functools

import jax
import jax.numpy as jnp
from jax import lax
from jax.experimental import pallas as pl
from jax.experimental.pallas import tpu as pltpu
from jax.experimental.pallas import tpu_sc as plsc

H = 128
NCELL = 12800
NCLU = 1408
NSTEP = 4
NW = 32
CS = 40
_F32 = jnp.float32

GY = ((12800, 'u', 'c', 0),
      (140800, 'u', 'c', 1),
      (2560, 'c', 'c', 2),
      (2560, 'c', 'c', 3),
      (1280, 'u', 'u', 0),
      (1280, 'u', 'u', 1))
GX = ((230400, 'c', 'c', 0),)


def _bases(spec):
    b, acc = [], 0
    for e in spec:
        b.append(acc)
        acc += e[0]
    return tuple(b), acc


GY_BASE, GY_E = _bases(GY)
GX_BASE, GX_E = _bases(GX)


@functools.cache
def _sc_mesh():
    return plsc.VectorSubcoreMesh(core_axis_name="c", subcore_axis_name="s")


def _mm_body(x_ref, w_ref, o_ref):
    o_ref[...] = jnp.dot(x_ref[...], w_ref[...], preferred_element_type=_F32)


def _mm(x, w, blk):
    m, k = x.shape
    n = w.shape[1]
    return pl.pallas_call(
        _mm_body,
        grid=(m // blk,),
        in_specs=[pl.BlockSpec((blk, k), lambda i: (i, 0)),
                  pl.BlockSpec((k, n), lambda i: (0, 0))],
        out_specs=pl.BlockSpec((blk, n), lambda i: (i, 0)),
        out_shape=jax.ShapeDtypeStruct((m, n), _F32),
    )(x, w)


def _g_body(w4_ref, wm_ref, o_ref):
    o_ref[0] = lax.dot_general(w4_ref[0], wm_ref[0], (((1,), (1,)), ((), ())),
                               preferred_element_type=_F32)


def _make_g(w4s, wm):
    return pl.pallas_call(
        _g_body,
        grid=(5,),
        in_specs=[pl.BlockSpec((1, H, H), lambda i: (i, 0, 0)),
                  pl.BlockSpec((1, 4 * H, H), lambda i: (i, 0, 0))],
        out_specs=pl.BlockSpec((1, H, 4 * H), lambda i: (i, 0, 0)),
        out_shape=jax.ShapeDtypeStruct((5, H, 4 * H), _F32),
    )(w4s, wm)


MLP_BLK = 1280


def _widx_y(i):
    w = ((i >= 10).astype(jnp.int32) + (i >= 120) + (i >= 122))
    w = jnp.where(i >= 125, 3, jnp.where(i >= 124, 2, w))
    return w


def _mlp_sel_body(x_ref, w2_ref, w3_ref, o_ref):
    y = jnp.maximum(jnp.dot(x_ref[...], w2_ref[0], preferred_element_type=_F32), 0.0)
    o_ref[...] = jnp.maximum(jnp.dot(y, w3_ref[0], preferred_element_type=_F32), 0.0)


def _mlp_y(y1, w2s, w3s):
    return pl.pallas_call(
        _mlp_sel_body,
        grid=(GY_E // MLP_BLK,),
        in_specs=[pl.BlockSpec((MLP_BLK, H), lambda i: (i, 0)),
                  pl.BlockSpec((1, H, H), lambda i: (_widx_y(i), 0, 0)),
                  pl.BlockSpec((1, H, H), lambda i: (_widx_y(i), 0, 0))],
        out_specs=pl.BlockSpec((MLP_BLK, H), lambda i: (i, 0)),
        out_shape=jax.ShapeDtypeStruct((GY_E, H), _F32),
    )(y1, w2s, w3s)


def _mlp_fix_body(x_ref, w2_ref, w3_ref, o_ref):
    y = jnp.maximum(jnp.dot(x_ref[...], w2_ref[...], preferred_element_type=_F32), 0.0)
    o_ref[...] = jnp.maximum(jnp.dot(y, w3_ref[...], preferred_element_type=_F32), 0.0)


def _mlp_x(y1, w2, w3):
    return pl.pallas_call(
        _mlp_fix_body,
        grid=(GX_E // MLP_BLK,),
        in_specs=[pl.BlockSpec((MLP_BLK, H), lambda i: (i, 0)),
                  pl.BlockSpec((H, H), lambda i: (0, 0)),
                  pl.BlockSpec((H, H), lambda i: (0, 0))],
        out_specs=pl.BlockSpec((MLP_BLK, H), lambda i: (i, 0)),
        out_shape=jax.ShapeDtypeStruct((GX_E, H), _F32),
    )(y1, w2, w3)


def _lstm_body(cc_ref, py_ref, px_ref, h_ref, c_ref, g_ref, whh_ref,
               ho_ref, co_ref):
    acc = cc_ref[...] + jnp.dot(h_ref[...], whh_ref[...], preferred_element_type=_F32)
    for t in range(4):
        p = py_ref[0, t] + py_ref[1, t]
        acc = acc + jnp.dot(p, g_ref[t], preferred_element_type=_F32)
    p = px_ref[0, 0] + px_ref[1, 0]
    acc = acc + jnp.dot(p, g_ref[4], preferred_element_type=_F32)
    i = jax.nn.sigmoid(acc[:, 0:H])
    f = jax.nn.sigmoid(acc[:, H:2 * H])
    g = jnp.tanh(acc[:, 2 * H:3 * H])
    o = jax.nn.sigmoid(acc[:, 3 * H:4 * H])
    cn = f * c_ref[...] + i * g
    ho_ref[...] = o * jnp.tanh(cn)
    co_ref[...] = cn


def _lstm(cellconst, py, px, h, c, g, whhT):
    blk = 1280
    return pl.pallas_call(
        _lstm_body,
        grid=(NCELL // blk,),
        in_specs=[pl.BlockSpec((blk, 4 * H), lambda i: (i, 0)),
                  pl.BlockSpec((2, 4, blk, H), lambda i: (0, 0, i, 0)),
                  pl.BlockSpec((2, 1, blk, H), lambda i: (0, 0, i, 0)),
                  pl.BlockSpec((blk, H), lambda i: (i, 0)),
                  pl.BlockSpec((blk, H), lambda i: (i, 0)),
                  pl.BlockSpec((5, H, 4 * H), lambda i: (0, 0, 0)),
                  pl.BlockSpec((H, 4 * H), lambda i: (0, 0))],
        out_specs=[pl.BlockSpec((blk, H), lambda i: (i, 0)),
                   pl.BlockSpec((blk, H), lambda i: (i, 0))],
        out_shape=[jax.ShapeDtypeStruct((NCELL, H), _F32),
                   jax.ShapeDtypeStruct((NCELL, H), _F32)],
    )(cellconst, py, px, h, c, g, whhT)


def _clu_body(hc_ref, pu_ref, w4_ref, o_ref):
    cm = jnp.dot(pu_ref[0, 0] + pu_ref[1, 0], w4_ref[2], preferred_element_type=_F32)
    cm = cm + jnp.dot(pu_ref[0, 1] + pu_ref[1, 1], w4_ref[3], preferred_element_type=_F32)
    o_ref[...] = jnp.tanh(hc_ref[...] + cm)


def _clu_update(h_clu, pu, w4s):
    return pl.pallas_call(
        _clu_body,
        in_specs=[pl.BlockSpec((NCLU, H), lambda: (0, 0)),
                  pl.BlockSpec((2, 2, NCLU, H), lambda: (0, 0, 0, 0)),
                  pl.BlockSpec((5, H, H), lambda: (0, 0, 0))],
        out_specs=pl.BlockSpec((NCLU, H), lambda: (0, 0)),
        out_shape=jax.ShapeDtypeStruct((NCLU, H), _F32),
    )(h_clu, pu, w4s)


def _logits_body(h0, h1, h2, h3, oe_ref, o_ref):
    oe = oe_ref[0]
    for t, hr in enumerate((h0, h1, h2, h3)):
        o_ref[t, 0] = lax.dot_general(hr[0], oe, (((1,), (1,)), ((), ())),
                                      preferred_element_type=_F32)


def _logits(hs, oe3):
    nb = 100
    hs3 = [h.reshape(128, nb, H) for h in hs]
    out = pl.pallas_call(
        _logits_body,
        grid=(128,),
        in_specs=[pl.BlockSpec((1, nb, H), lambda i: (i, 0, 0)) for _ in range(4)]
        + [pl.BlockSpec((1, 11, H), lambda i: (i, 0, 0))],
        out_specs=pl.BlockSpec((4, 1, nb, 11), lambda i: (0, i, 0, 0)),
        out_shape=jax.ShapeDtypeStruct((4, 128, nb, 11), _F32),
    )(*hs3, oe3)
    return out.reshape(4, NCELL, 11)


def _sc_gather(spec, bases, etot, tcell, tclu, gidxa, gidxb):
    pws = tuple(e[0] // NW for e in spec)
    pwmax = max(pws)
    GCH = 128

    @functools.partial(
        pl.kernel,
        out_type=jax.ShapeDtypeStruct((etot, H), _F32),
        mesh=_sc_mesh(),
        scratch_types=[
            pltpu.VMEM((pwmax,), jnp.int32),
            pltpu.VMEM((pwmax,), jnp.int32),
            pltpu.VMEM((2, GCH, H), _F32),
            pltpu.VMEM((2, GCH, H), _F32),
            pltpu.SemaphoreType.DMA,
            pltpu.SemaphoreType.DMA,
            pltpu.SemaphoreType.DMA,
        ],
    )
    def k(tc_ref, tu_ref, ia_ref, ib_ref, y_ref, iav, ibv, ra, rb,
          sema, semb, semw):
        w = lax.axis_index("s") * 2 + lax.axis_index("c")
        for s in range(len(spec)):
            pw = pws[s]
            wbase = bases[s] + w * pw
            ta = tu_ref if spec[s][1] == 'u' else tc_ref
            tb = tu_ref if spec[s][2] == 'u' else tc_ref
            pltpu.sync_copy(ia_ref.at[pl.ds(wbase, pw)], iav.at[pl.ds(0, pw)])
            pltpu.sync_copy(ib_ref.at[pl.ds(wbase, pw)], ibv.at[pl.ds(0, pw)])

            def half(off, u, gc, ta=ta, tb=tb):
                cpa = pltpu.async_copy(ta.at[iav.at[pl.ds(off, gc)]],
                                       ra.at[u, pl.ds(0, gc)], sema)
                cpb = pltpu.async_copy(tb.at[ibv.at[pl.ds(off, gc)]],
                                       rb.at[u, pl.ds(0, gc)], semb)
                return cpa, cpb

            def work(off, u, gc, wbase=wbase):
                def row(r, _):
                    for kk in range(8):
                        a = ra[u, r, pl.ds(kk * 16, 16)]
                        b = rb[u, r, pl.ds(kk * 16, 16)]
                        ra[u, r, pl.ds(kk * 16, 16)] = jnp.maximum(a + b, 0.0)
                    return 0

                lax.fori_loop(0, gc, row, 0)
                return pltpu.async_copy(ra.at[u, pl.ds(0, gc)],
                                        y_ref.at[pl.ds(wbase + off, gc)],
                                        semw)

            def single(off, gc):
                ca, cb = half(off, 0, gc)
                ca.wait()
                cb.wait()
                work(off, 0, gc).wait()

            nfull = pw // GCH
            rem = pw % GCH
            if nfull >= 2:
                def pair(jj, _):
                    j = jj * 2
                    ca0, cb0 = half(j * GCH, 0, GCH)
                    ca1, cb1 = half((j + 1) * GCH, 1, GCH)
                    ca0.wait()
                    cb0.wait()
                    w0 = work(j * GCH, 0, GCH)
                    ca1.wait()
                    cb1.wait()
                    w1 = work((j + 1) * GCH, 1, GCH)
                    w0.wait()
                    w1.wait()
                    return 0

                lax.fori_loop(0, nfull // 2, pair, 0)
            if nfull % 2 == 1:
                single((nfull - 1) * GCH, GCH)
            if rem:
                single(nfull * GCH, rem)

    return k(tcell, tclu, gidxa, gidxb)


def _sc_scatter(spec, bases, ncslot, nuslot, y3, gdst, zc):
    pws = tuple(e[0] // NW for e in spec)
    out_type = [jax.ShapeDtypeStruct((2, ncslot, NCELL, H), _F32)]
    if nuslot:
        out_type.append(jax.ShapeDtypeStruct((2, nuslot, NCLU, H), _F32))

    @functools.partial(
        pl.kernel,
        out_type=out_type,
        mesh=_sc_mesh(),
        scratch_types=[
            pltpu.VMEM((1, 2 * CS), jnp.int32),
            pltpu.VMEM((1, CS), jnp.int32),
            pltpu.VMEM((2 * CS, H), _F32),
            pltpu.VMEM_SHARED((NCELL, H), _F32),
            pltpu.SemaphoreType.DMA,
            pltpu.SemaphoreType.DMA,
            pltpu.SemaphoreType.DMA,
        ],
    )
    def k(y_ref, gd_ref, zc_ref, oc_ref, *rest):
        if nuslot:
            ou_ref = rest[0]
            rest = rest[1:]
        idx8, idx4, ybuf, accum, lsem, isem, ssem = rest
        c = lax.axis_index("c")
        s = lax.axis_index("s")

        for seg in range(len(spec)):
            slot_cell = spec[seg][2] == 'c'
            nrows = NCELL if slot_cell else NCLU
            rpt = nrows // 16
            r0 = s * rpt
            pw = pws[seg]
            cs = 2 * CS if pw % (2 * CS) == 0 else CS
            nch = pw // cs
            off0 = bases[seg] + c * (spec[seg][0] // 2) + s * pw
            pltpu.sync_copy(zc_ref.at[pl.ds(0, rpt)], accum.at[pl.ds(r0, rpt)])
            plsc.subcore_barrier()

            def chunk(j, _, off0=off0, cs=cs):
                o = off0 + j * cs
                iv = idx8 if cs == 2 * CS else idx4
                ly = pltpu.async_copy(y_ref.at[pl.ds(o, cs)],
                                      ybuf.at[pl.ds(0, cs)], lsem)
                l0 = pltpu.async_copy(gd_ref.at[pl.ds(o, cs)], iv.at[0], isem)
                ly.wait()
                l0.wait()
                pltpu.sync_copy(ybuf.at[pl.ds(0, cs)], accum.at[iv.at[0]],
                                add=True)
                return 0

            if nch == 1:
                chunk(0, 0)
            else:
                lax.fori_loop(0, nch, chunk, 0)
            plsc.subcore_barrier()
            slot = spec[seg][3]
            if slot_cell:
                pltpu.sync_copy(accum.at[pl.ds(r0, rpt)],
                                oc_ref.at[c, slot, pl.ds(r0, rpt)])
            else:
                pltpu.sync_copy(accum.at[pl.ds(r0, rpt)],
                                ou_ref.at[c, slot, pl.ds(r0, rpt)])
            plsc.subcore_barrier()

    return k(y3, gdst, zc)


def kernel(cell_x, cluster_x, output_embeddings, params, contains_src,
           contains_dst, may_src, may_dst, lt_edges, gt_edges, diff_edges,
           clt_edges, cgt_edges):
    order = ('contains', 'may_contain', 'lt', 'gt', 'diff')
    w1a = {t: params[t]['W1'][:H] for t in order}
    w1b = {t: params[t]['W1'][H:] for t in order}
    wcat_cell = jnp.concatenate(
        [w1b['contains'], w1b['may_contain'], w1a['lt'], w1b['lt'],
         w1a['gt'], w1b['gt'], w1a['diff'], w1b['diff']], axis=1)
    wcat_clu = jnp.concatenate(
        [w1a['contains'], w1a['may_contain'], w1a['lt'], w1b['lt'],
         w1a['gt'], w1b['gt']], axis=1)
    w2s = jnp.stack([params[t]['W2'] for t in order[:4]])
    w3s = jnp.stack([params[t]['W3'] for t in order[:4]])
    w4s = jnp.stack([params[t]['W4'] for t in order])
    wih = params['Wih']
    wihxT = wih[:, :H].T
    wm = wih[:, H:].reshape(4 * H, 5, H).transpose(1, 0, 2)
    whhT = params['Whh'].T

    i32 = jnp.int32
    gay = jnp.concatenate([
        contains_src.astype(i32) * 6, may_src.astype(i32) * 6 + 1,
        lt_edges[0].astype(i32) * 8 + 2, gt_edges[0].astype(i32) * 8 + 4,
        clt_edges[0].astype(i32) * 6 + 2, cgt_edges[0].astype(i32) * 6 + 4])
    gby = jnp.concatenate([
        contains_dst.astype(i32) * 8, may_dst.astype(i32) * 8 + 1,
        lt_edges[1].astype(i32) * 8 + 3, gt_edges[1].astype(i32) * 8 + 5,
        clt_edges[1].astype(i32) * 6 + 3, cgt_edges[1].astype(i32) * 6 + 5])
    gdy = jnp.concatenate([
        contains_dst.astype(i32), may_dst.astype(i32), lt_edges[1].astype(i32),
        gt_edges[1].astype(i32), clt_edges[1].astype(i32),
        cgt_edges[1].astype(i32)])
    gax = diff_edges[0].astype(i32) * 8 + 6
    gbx = diff_edges[1].astype(i32) * 8 + 7
    gdx = diff_edges[1].astype(i32)

    cellconst = _mm(cell_x, wihxT, 1280)
    g = _make_g(w4s, wm)
    zc = jnp.zeros((NCELL // 16, H), _F32)

    h_cell = cell_x
    h_clu = cluster_x
    rnn_h = jnp.zeros((NCELL, H), _F32)
    rnn_c = jnp.zeros((NCELL, H), _F32)
    hs = []
    for _ in range(NSTEP):
        tcell = _mm(h_cell, wcat_cell, 1280).reshape(NCELL * 8, H)
        tclu = _mm(h_clu, wcat_clu, NCLU).reshape(NCLU * 6, H)
        y1y = _sc_gather(GY, GY_BASE, GY_E, tcell, tclu, gay, gby)
        y1x = _sc_gather(GX, GX_BASE, GX_E, tcell, tclu, gax, gbx)
        y3y = _mlp_y(y1y, w2s, w3s)
        y3x = _mlp_x(y1x, params['diff']['W2'], params['diff']['W3'])
        py, pu = _sc_scatter(GY, GY_BASE, 4, 2, y3y, gdy, zc)
        (px,) = _sc_scatter(GX, GX_BASE, 1, 0, y3x, gdx, zc)
        rnn_h, rnn_c = _lstm(cellconst, py, px, rnn_h, rnn_c, g, whhT)
        h_cell = rnn_h
        h_clu = _clu_update(h_clu, pu, w4s)
        hs.append(h_cell)

    oe3 = output_embeddings.reshape(128, 11, H)
    return _logits(hs, oe3)
